# Initial kernel scaffold; baseline (speedup 1.0000x reference)
import jax
import jax.numpy as jnp
from jax.experimental import pallas as pl


def kernel(x, edge_index, pos, y, lin_W, lift_W, lift_b, hid_W, hid_b, low_W, low_b, bn0_g, bn0_b, bn1_g, bn1_b, bn_g, bn_b):
    raise NotImplementedError("write your pallas kernel here")


# speedup vs baseline: 5.0931x; 5.0931x over previous
import functools

import jax
import jax.numpy as jnp
from jax import lax
from jax.experimental import pallas as pl
from jax.experimental.pallas import tpu as pltpu
from jax.experimental.pallas import tpu_sc as plsc

N = 10000
C = 128
E = 320000
M = E + N
NW = 32
CH = 128
EPW = 10368
M_PAD = NW * EPW
N_PAD = M_PAD - M
NROWS = N + 8
ACC_ROWS = 10112
RPS = ACC_ROWS // 16
DEGW = 16
BLK = 2048
NBLK = M_PAD // BLK
NCHUNK = EPW // CH

_mesh = plsc.VectorSubcoreMesh(core_axis_name="c", subcore_axis_name="s")


def _elu(u):
    return jnp.where(u > 0, u, jnp.exp(jnp.minimum(u, 0.0)) - 1.0)


def _tc_prep_body(pos_ref, y_ref, x_ref, wps_ref, wpd_ref, wys_ref, wyd_ref,
                  lin_ref, a_ref, b_ref, xw_ref):
    p = pos_ref[...]
    yv = y_ref[...]
    f32 = jnp.float32
    a_ref[0:N, :] = (jnp.dot(p, wps_ref[...], preferred_element_type=f32)
                     + jnp.dot(yv, wys_ref[...], preferred_element_type=f32))
    b_ref[0:N, :] = (jnp.dot(p, wpd_ref[...], preferred_element_type=f32)
                     + jnp.dot(yv, wyd_ref[...], preferred_element_type=f32))
    a_ref[N:NROWS, :] = jnp.zeros((NROWS - N, C), f32)
    b_ref[N:NROWS, :] = jnp.zeros((NROWS - N, C), f32)
    xw_ref[...] = jnp.dot(x_ref[...], lin_ref[...], preferred_element_type=f32)


def _tc_prep(pos8, y, x, wps, wpd, wys, wyd, lin_W):
    sds = jax.ShapeDtypeStruct
    return pl.pallas_call(
        _tc_prep_body,
        out_shape=[sds((NROWS, C), jnp.float32),
                   sds((NROWS, C), jnp.float32),
                   sds((N, C), jnp.float32)],
    )(pos8, y, x, wps, wpd, wys, wyd, lin_W)


@functools.partial(
    pl.kernel,
    out_type=[jax.ShapeDtypeStruct((M_PAD, C), jnp.float32),
              jax.ShapeDtypeStruct((2, ACC_ROWS, DEGW), jnp.float32)],
    mesh=_mesh,
    scratch_types=[
        pltpu.VMEM((CH,), jnp.int32),
        pltpu.VMEM((CH,), jnp.int32),
        pltpu.VMEM((CH, C), jnp.float32),
        pltpu.VMEM((CH, C), jnp.float32),
        pltpu.VMEM((CH, DEGW), jnp.float32),
        pltpu.VMEM_SHARED((ACC_ROWS, DEGW), jnp.float32),
        pltpu.SemaphoreType.DMA,
        pltpu.SemaphoreType.DMA,
    ],
)
def _sc_gather(a_hbm, b_hbm, src_hbm, dst_hbm, zdeg_hbm, ones_hbm,
               z0_hbm, d_out,
               idx_s, idx_d, abuf, bbuf, onesbuf, dacc, sem_a, sem_b):
    cid = lax.axis_index("c")
    sid = lax.axis_index("s")
    wid = sid * 2 + cid
    base0 = wid * EPW

    pltpu.sync_copy(zdeg_hbm, dacc.at[pl.ds(sid * RPS, RPS)])
    pltpu.sync_copy(ones_hbm, onesbuf)
    plsc.subcore_barrier()

    @pl.loop(0, NCHUNK)
    def _(ch):
        base = base0 + ch * CH
        pltpu.sync_copy(src_hbm.at[pl.ds(base, CH)], idx_s)
        pltpu.sync_copy(dst_hbm.at[pl.ds(base, CH)], idx_d)
        cpa = pltpu.async_copy(a_hbm.at[idx_s], abuf, sem_a)
        cpb = pltpu.async_copy(b_hbm.at[idx_d], bbuf, sem_b)
        cpa.wait()
        cpb.wait()

        @pl.loop(0, CH)
        def _(r):
            for g in range(C // 16):
                sl = pl.ds(g * 16, 16)
                abuf[r, sl] = abuf[r, sl] + bbuf[r, sl]

        pltpu.sync_copy(abuf, z0_hbm.at[pl.ds(base, CH)])
        pltpu.sync_copy(onesbuf, dacc.at[idx_s], add=True)

    plsc.subcore_barrier()
    pltpu.sync_copy(dacc.at[pl.ds(sid * RPS, RPS)],
                    d_out.at[cid, pl.ds(sid * RPS, RPS)])


def _tc_stats_body(z_ref, stats_ref, acc_ref):
    i = pl.program_id(0)

    @pl.when(i == 0)
    def _():
        acc_ref[...] = jnp.zeros((2, C), jnp.float32)

    z = z_ref[...]
    acc_ref[0:1, :] += jnp.sum(z, axis=0, keepdims=True)
    acc_ref[1:2, :] += jnp.sum(z * z, axis=0, keepdims=True)

    @pl.when(i == NBLK - 1)
    def _():
        stats_ref[...] = acc_ref[...]


def _tc_stats(z0):
    return pl.pallas_call(
        _tc_stats_body,
        grid=(NBLK,),
        in_specs=[pl.BlockSpec((BLK, C), lambda i: (i, 0))],
        out_specs=pl.BlockSpec((2, C), lambda i: (0, 0)),
        out_shape=jax.ShapeDtypeStruct((2, C), jnp.float32),
        scratch_shapes=[pltpu.VMEM((2, C), jnp.float32)],
    )(z0)


def _tc_layer1_body(z0_ref, sc_ref, sh_ref, w_ref, z1_ref, stats_ref, acc_ref):
    i = pl.program_id(0)

    @pl.when(i == 0)
    def _():
        acc_ref[...] = jnp.zeros((2, C), jnp.float32)

    u = z0_ref[...] * sc_ref[...] + sh_ref[...]
    h = _elu(u)
    rows = i * BLK + lax.broadcasted_iota(jnp.int32, (BLK, 1), 0)
    h = jnp.where(rows < M, h, 0.0)
    z1 = jnp.dot(h, w_ref[...], preferred_element_type=jnp.float32)
    z1_ref[...] = z1
    acc_ref[0:1, :] += jnp.sum(z1, axis=0, keepdims=True)
    acc_ref[1:2, :] += jnp.sum(z1 * z1, axis=0, keepdims=True)

    @pl.when(i == NBLK - 1)
    def _():
        stats_ref[...] = acc_ref[...]


def _tc_layer1(z0, sc0, sh0, hid_W):
    sds = jax.ShapeDtypeStruct
    return pl.pallas_call(
        _tc_layer1_body,
        grid=(NBLK,),
        in_specs=[pl.BlockSpec((BLK, C), lambda i: (i, 0)),
                  pl.BlockSpec((1, C), lambda i: (0, 0)),
                  pl.BlockSpec((1, C), lambda i: (0, 0)),
                  pl.BlockSpec((C, C), lambda i: (0, 0))],
        out_specs=[pl.BlockSpec((BLK, C), lambda i: (i, 0)),
                   pl.BlockSpec((2, C), lambda i: (0, 0))],
        out_shape=[sds((M_PAD, C), jnp.float32), sds((2, C), jnp.float32)],
        scratch_shapes=[pltpu.VMEM((2, C), jnp.float32)],
    )(z0, sc0, sh0, hid_W)


def _tc_layer2_body(z1_ref, sc_ref, sh_ref, w_ref, b_ref, ew_ref):
    u = z1_ref[...] * sc_ref[...] + sh_ref[...]
    h = _elu(u)
    ew_ref[...] = (jnp.dot(h, w_ref[...], preferred_element_type=jnp.float32)
                   + b_ref[...])


def _tc_layer2(z1, sc1, sh1, low_W, low_b):
    return pl.pallas_call(
        _tc_layer2_body,
        grid=(NBLK,),
        in_specs=[pl.BlockSpec((BLK, C), lambda i: (i, 0)),
                  pl.BlockSpec((1, C), lambda i: (0, 0)),
                  pl.BlockSpec((1, C), lambda i: (0, 0)),
                  pl.BlockSpec((C, C), lambda i: (0, 0)),
                  pl.BlockSpec((1, C), lambda i: (0, 0))],
        out_specs=pl.BlockSpec((BLK, C), lambda i: (i, 0)),
        out_shape=jax.ShapeDtypeStruct((M_PAD, C), jnp.float32),
    )(z1, sc1, sh1, low_W, low_b)


@functools.partial(
    pl.kernel,
    out_type=jax.ShapeDtypeStruct((2, ACC_ROWS, C), jnp.float32),
    mesh=_mesh,
    scratch_types=[
        pltpu.VMEM((CH,), jnp.int32),
        pltpu.VMEM((CH,), jnp.int32),
        pltpu.VMEM((CH, C), jnp.float32),
        pltpu.VMEM((CH, C), jnp.float32),
        pltpu.VMEM_SHARED((ACC_ROWS, C), jnp.float32),
        pltpu.SemaphoreType.DMA,
    ],
)
def _sc_scatter(ew_hbm, x_hbm, src_hbm, dst_hbm, zrow_hbm,
                s_out, idx_s, idx_d, ewbuf, xbuf, sacc, sem):
    cid = lax.axis_index("c")
    sid = lax.axis_index("s")
    wid = sid * 2 + cid
    base0 = wid * EPW

    pltpu.sync_copy(zrow_hbm, sacc.at[pl.ds(sid * RPS, RPS)])
    plsc.subcore_barrier()

    @pl.loop(0, NCHUNK)
    def _(ch):
        base = base0 + ch * CH
        pltpu.sync_copy(src_hbm.at[pl.ds(base, CH)], idx_s)
        pltpu.sync_copy(dst_hbm.at[pl.ds(base, CH)], idx_d)
        pltpu.sync_copy(ew_hbm.at[pl.ds(base, CH)], ewbuf)
        pltpu.async_copy(x_hbm.at[idx_s], xbuf, sem).wait()

        @pl.loop(0, CH)
        def _(r):
            for g in range(C // 16):
                sl = pl.ds(g * 16, 16)
                ewbuf[r, sl] = ewbuf[r, sl] * xbuf[r, sl]

        pltpu.sync_copy(ewbuf, sacc.at[idx_d], add=True)

    plsc.subcore_barrier()
    pltpu.sync_copy(sacc.at[pl.ds(sid * RPS, RPS)],
                    s_out.at[cid, pl.ds(sid * RPS, RPS)])


def _tc_final_body(xw_ref, s_ref, d_ref, g_ref, b_ref, o_ref):
    s = s_ref[0, 0:N, :] + s_ref[1, 0:N, :]
    deg = d_ref[0, 0:N, 0:1] + d_ref[1, 0:N, 0:1]
    v = xw_ref[...] + s / deg
    m = jnp.mean(v, axis=0, keepdims=True)
    var = jnp.mean(v * v, axis=0, keepdims=True) - m * m
    u = g_ref[...] * (v - m) * lax.rsqrt(var + 1e-5) + b_ref[...]
    o_ref[...] = _elu(u)


def _tc_final(xw, s_part, d_part, bn_g, bn_b):
    return pl.pallas_call(
        _tc_final_body,
        out_shape=jax.ShapeDtypeStruct((N, C), jnp.float32),
    )(xw, s_part, d_part, bn_g, bn_b)


def kernel(x, edge_index, pos, y, lin_W, lift_W, lift_b, hid_W, hid_b,
           low_W, low_b, bn0_g, bn0_b, bn1_g, bn1_b, bn_g, bn_b):
    f32 = jnp.float32
    wps = jnp.concatenate([lift_W[0:3], jnp.zeros((5, C), f32)], axis=0)
    wpd = jnp.concatenate([lift_W[3:6], jnp.zeros((5, C), f32)], axis=0)
    wys = lift_W[6:6 + C]
    wyd = lift_W[6 + C:]
    pos8 = jnp.concatenate([pos, jnp.zeros((N, 5), f32)], axis=1)

    ar = jnp.arange(N, dtype=jnp.int32)
    sent = jnp.full((N_PAD,), N, jnp.int32)
    src = jnp.concatenate([edge_index[0].astype(jnp.int32), ar, sent])
    dst = jnp.concatenate([edge_index[1].astype(jnp.int32), ar, sent])

    a_tbl, b_tbl, xw = _tc_prep(pos8, y, x, wps, wpd, wys, wyd, lin_W)
    x_tbl = jnp.concatenate([x, jnp.zeros((NROWS - N, C), f32)], axis=0)

    zdeg = jnp.zeros((RPS, DEGW), f32)
    ones = jnp.ones((CH, DEGW), f32)
    z0, d_part = _sc_gather(a_tbl, b_tbl, src, dst, zdeg, ones)

    stats0 = _tc_stats(z0)
    m0 = stats0[0] / M
    v0 = stats0[1] / M - m0 * m0
    sc0 = (bn0_g * lax.rsqrt(v0 + 1e-5)).reshape(1, C)
    sh0 = (bn0_b - m0 * bn0_g * lax.rsqrt(v0 + 1e-5)).reshape(1, C)

    z1, stats1 = _tc_layer1(z0, sc0, sh0, hid_W)
    m1 = stats1[0] / M
    v1 = stats1[1] / M - m1 * m1
    sc1 = (bn1_g * lax.rsqrt(v1 + 1e-5)).reshape(1, C)
    sh1 = (bn1_b - m1 * bn1_g * lax.rsqrt(v1 + 1e-5)).reshape(1, C)

    ew = _tc_layer2(z1, sc1, sh1, low_W, low_b.reshape(1, C))

    zrow = jnp.zeros((RPS, C), f32)
    s_part = _sc_scatter(ew, x_tbl, src, dst, zrow)

    return _tc_final(xw, s_part, d_part, bn_g.reshape(1, C), bn_b.reshape(1, C))
